# baseline (speedup 1.0000x reference)
import jax
import jax.numpy as jnp
from jax.experimental import pallas as pl
from jax.experimental.pallas import tpu as pltpu

_CR = 8
_NBUF = 4


def _bias_add_kernel(x_ref, b_ref, o_hbm, obufs, sems):
    i = pl.program_id(0)
    n = pl.num_programs(0)
    b = jax.lax.rem(i, _NBUF)

    def out_copy(chunk, slot):
        return pltpu.make_async_copy(
            obufs.at[slot], o_hbm.at[pl.ds(chunk * _CR, _CR), :], sems.at[slot])

    @pl.when(i >= _NBUF)
    def _():
        out_copy(i - _NBUF, b).wait()

    obufs[b] = x_ref[...] + b_ref[...]
    out_copy(i, b).start()

    @pl.when(i == n - 1)
    def _():
        for k in range(_NBUF):
            slot = jax.lax.rem(i - k, _NBUF)
            out_copy(i - k, slot).wait()


def kernel(user_emb, loc_logits, user_loc_weights, loc_bias):
    B, L = loc_logits.shape
    bias2d = loc_bias.reshape(1, L)
    out = pl.pallas_call(
        _bias_add_kernel,
        grid=(B // _CR,),
        in_specs=[
            pl.BlockSpec((_CR, L), lambda i: (i, 0)),
            pl.BlockSpec((1, L), lambda i: (0, 0)),
        ],
        out_specs=pl.BlockSpec(memory_space=pltpu.MemorySpace.HBM),
        out_shape=jax.ShapeDtypeStruct((B, L), jnp.float32),
        scratch_shapes=[
            pltpu.VMEM((_NBUF, _CR, L), jnp.float32),
            pltpu.SemaphoreType.DMA((_NBUF,)),
        ],
        compiler_params=pltpu.CompilerParams(vmem_limit_bytes=60 * 1024 * 1024),
    )(loc_logits, bias2d)
    return out


# speedup vs baseline: 1.0039x; 1.0039x over previous
import jax
import jax.numpy as jnp
from jax.experimental import pallas as pl
from jax.experimental.pallas import tpu as pltpu

_R = 32


def _bias_add_kernel(x_ref, b_ref, o_ref):
    o_ref[...] = x_ref[...] + b_ref[...]


def kernel(user_emb, loc_logits, user_loc_weights, loc_bias):
    B, L = loc_logits.shape
    bias2d = loc_bias.reshape(1, L)
    out = pl.pallas_call(
        _bias_add_kernel,
        grid=(B // _R,),
        in_specs=[
            pl.BlockSpec((_R, L), lambda i: (i, 0)),
            pl.BlockSpec((1, L), lambda i: (0, 0)),
        ],
        out_specs=pl.BlockSpec((_R, L), lambda i: (i, 0)),
        out_shape=jax.ShapeDtypeStruct((B, L), jnp.float32),
        compiler_params=pltpu.CompilerParams(vmem_limit_bytes=60 * 1024 * 1024),
    )(loc_logits, bias2d)
    return out
